# baseline (speedup 1.0000x reference)
import functools

import jax
import jax.numpy as jnp
from jax import lax
from jax.experimental import pallas as pl
from jax.experimental.pallas import tpu as pltpu
from jax.experimental.pallas import tpu_sc as plsc

_N = 50000
_NW = 32
_WPT = 1568
_BLK = 32
_LEVELS = (
    (56, 64, 0.25, 3, 32),
    (28, 128, 0.125, 67, 32),
    (14, 256, 0.0625, 195, 16),
    (7, 512, 1.0 / 32.0, 451, 16),
)


def _scratch_types():
    t = [
        pltpu.VMEM((_WPT,), jnp.float32),
        pltpu.VMEM((_WPT,), jnp.float32),
        pltpu.VMEM((_WPT,), jnp.float32),
        pltpu.VMEM((_WPT,), jnp.float32),
        pltpu.VMEM((_WPT,), jnp.float32),
        pltpu.VMEM((_BLK, 8, 128), jnp.float32),
    ]
    for (_, dim, _, _, sub) in _LEVELS:
        t.extend([pltpu.VMEM((sub, dim), jnp.float32)] * 4)
        t.extend([pltpu.VMEM((sub,), jnp.int32)] * 4)
        t.extend([pltpu.VMEM((sub,), jnp.float32)] * 4)
    t.append(pltpu.SemaphoreType.DMA)
    return t


def _worker_id():
    return lax.axis_index("s") * 2 + lax.axis_index("c")


def _sc_body(x_hbm, y_hbm, z_hbm, t0, t1, t2, t3, out, *scr):
    xv, yv, zv, hv, wv, st = scr[:6]
    per_level = []
    k = 6
    for _ in _LEVELS:
        per_level.append(scr[k:k + 12])
        k += 12
    sem = scr[k]

    wid = _worker_id()
    base = jnp.minimum(wid * _WPT, _N - _WPT)

    pltpu.sync_copy(x_hbm.at[pl.ds(base, _WPT)], xv)
    pltpu.sync_copy(y_hbm.at[pl.ds(base, _WPT)], yv)
    pltpu.sync_copy(z_hbm.at[pl.ds(base, _WPT)], zv)

    lane = lax.iota(jnp.int32, 16)
    is0 = lane == 0
    is1 = lane == 1

    def hw_body(c, carry):
        s = c * 16
        xx = xv[pl.ds(s, 16)]
        yy = yv[pl.ds(s, 16)]
        zz = zv[pl.ds(s, 16)]
        nz = -zz
        hh = 250.0 * (-yy) / nz + 112.0
        ww = 250.0 * xx / nz + 112.0
        hv[pl.ds(s, 16)] = jnp.minimum(jnp.maximum(hh, 0.0), 223.0)
        wv[pl.ds(s, 16)] = jnp.minimum(jnp.maximum(ww, 0.0), 223.0)
        return carry

    lax.fori_loop(0, _WPT // 16, hw_body, 0)

    tabs = (t0, t1, t2, t3)

    def blk_body(b, carry):
        pb = b * _BLK

        for g in range(_BLK // 16):
            xx = xv[pl.ds(pb + g * 16, 16)]
            yy = yv[pl.ds(pb + g * 16, 16)]
            zz = zv[pl.ds(pb + g * 16, 16)]
            for j in range(16):
                cvec = jnp.where(is0, xx[j], jnp.where(is1, yy[j], zz[j]))
                st[g * 16 + j, 0, pl.ds(0, 16)] = cvec

        for lvl, (H, dim, scale, col0, sub) in enumerate(_LEVELS):
            tab = tabs[lvl]
            q11, q21, q12, q22, i11, i21, i12, i22, w11, w21, w12, w22 = \
                per_level[lvl]
            for sb in range(_BLK // sub):
                ps = pb + sb * sub

                for c in range(sub // 16):
                    s = ps + c * 16
                    t = c * 16
                    hx = hv[pl.ds(s, 16)] * scale
                    wy = wv[pl.ds(s, 16)] * scale
                    x1i = hx.astype(jnp.int32)
                    x1f = x1i.astype(jnp.float32)
                    x2f = jnp.where(x1f == hx, x1f, x1f + 1.0)
                    x2i = jnp.minimum(x2f.astype(jnp.int32), H - 1)
                    y1i = wy.astype(jnp.int32)
                    y1f = y1i.astype(jnp.float32)
                    y2f = jnp.where(y1f == wy, y1f, y1f + 1.0)
                    y2i = jnp.minimum(y2f.astype(jnp.int32), H - 1)
                    dx2 = x2f - hx
                    dx1 = hx - x1f
                    dy2 = y2f - wy
                    dy1 = wy - y1f
                    i11[pl.ds(t, 16)] = x1i * H + y1i
                    i21[pl.ds(t, 16)] = x2i * H + y1i
                    i12[pl.ds(t, 16)] = x1i * H + y2i
                    i22[pl.ds(t, 16)] = x2i * H + y2i
                    w11[pl.ds(t, 16)] = dx2 * dy2
                    w21[pl.ds(t, 16)] = dx1 * dy2
                    w12[pl.ds(t, 16)] = dx2 * dy1
                    w22[pl.ds(t, 16)] = dx1 * dy1

                c1 = pltpu.async_copy(tab.at[i11], q11, sem)
                c2 = pltpu.async_copy(tab.at[i21], q21, sem)
                c3 = pltpu.async_copy(tab.at[i12], q12, sem)
                c4 = pltpu.async_copy(tab.at[i22], q22, sem)
                c1.wait()
                c2.wait()
                c3.wait()
                c4.wait()

                for g in range(sub // 16):
                    a16 = w11[pl.ds(g * 16, 16)]
                    b16 = w21[pl.ds(g * 16, 16)]
                    c16 = w12[pl.ds(g * 16, 16)]
                    d16 = w22[pl.ds(g * 16, 16)]
                    for j in range(16):
                        a, bw, cw, dw = a16[j], b16[j], c16[j], d16[j]
                        p = g * 16 + j
                        rp = sb * sub + g * 16 + j

                        def ch_body(kk, carry3, p=p, rp=rp, a=a, bw=bw,
                                    cw=cw, dw=dw, q11=q11, q21=q21,
                                    q12=q12, q22=q22, col0=col0):
                            d = pl.ds(kk * 16, 16)
                            v = (a * q11[p, d] + bw * q21[p, d]
                                 + cw * q12[p, d] + dw * q22[p, d])
                            cc = col0 + kk * 16
                            st[rp, lax.shift_right_logical(cc, 7),
                               pl.ds(lax.bitwise_and(cc, 127), 16)] = v
                            return carry3

                        lax.fori_loop(0, dim // 16, ch_body, 0)

        pltpu.sync_copy(st, out.at[pl.ds(base + pb, _BLK)])
        return carry

    lax.fori_loop(0, _WPT // _BLK, blk_body, 0)


@functools.cache
def _build_sc_kernel():
    mesh = plsc.VectorSubcoreMesh(
        core_axis_name="c", subcore_axis_name="s", num_cores=2, num_subcores=16
    )
    return functools.partial(
        pl.kernel,
        out_type=jax.ShapeDtypeStruct((_N, 8, 128), jnp.float32),
        mesh=mesh,
        scratch_types=_scratch_types(),
        compiler_params=pltpu.CompilerParams(use_tc_tiling_on_sc=False),
    )(_sc_body)


_FB = 1000


def _tc_finish_body(big_ref, out_ref):
    v = big_ref[...]
    for j in range(7):
        out_ref[:, pl.ds(j * 128, 128)] = v[:, j, :]
    out_ref[:, pl.ds(896, 67)] = v[:, 7, :67]


@functools.cache
def _build_tc_finisher():
    return pl.pallas_call(
        _tc_finish_body,
        grid=(_N // _FB,),
        in_specs=[pl.BlockSpec((_FB, 8, 128), lambda i: (i, 0, 0))],
        out_specs=pl.BlockSpec((_FB, 963), lambda i: (i, 0)),
        out_shape=jax.ShapeDtypeStruct((_N, 963), jnp.float32),
    )


def kernel(inputs, img_feat0, img_feat1, img_feat2, img_feat3):
    x = inputs[:, 0]
    y = inputs[:, 1]
    z = inputs[:, 2]
    t0 = img_feat0.reshape(56 * 56, 64)
    t1 = img_feat1.reshape(28 * 28, 128)
    t2 = img_feat2.reshape(14 * 14, 256)
    t3 = img_feat3.reshape(7 * 7, 512)
    big = _build_sc_kernel()(x, y, z, t0, t1, t2, t3)
    return _build_tc_finisher()(big)


# speedup vs baseline: 1.0038x; 1.0038x over previous
import functools

import jax
import jax.numpy as jnp
from jax import lax
from jax.experimental import pallas as pl
from jax.experimental.pallas import tpu as pltpu
from jax.experimental.pallas import tpu_sc as plsc

_N = 50000
_NW = 32
_WPT = 1568
_BLK = 32
_LEVELS = (
    (56, 64, 0.25, 0, 32),
    (28, 128, 0.125, 64, 32),
    (14, 256, 0.0625, 192, 16),
    (7, 512, 1.0 / 32.0, 448, 16),
)


def _scratch_types():
    t = [
        pltpu.VMEM((_WPT,), jnp.float32),
        pltpu.VMEM((_WPT,), jnp.float32),
        pltpu.VMEM((_WPT,), jnp.float32),
        pltpu.VMEM((_WPT,), jnp.float32),
        pltpu.VMEM((_WPT,), jnp.float32),
        pltpu.VMEM((_BLK, 8, 128), jnp.float32),
    ]
    for (_, dim, _, _, sub) in _LEVELS:
        t.extend([pltpu.VMEM((sub, dim), jnp.float32)] * 4)
        t.extend([pltpu.VMEM((sub,), jnp.int32)] * 4)
        t.extend([pltpu.VMEM((sub,), jnp.float32)] * 4)
    t.append(pltpu.SemaphoreType.DMA)
    return t


def _worker_id():
    return lax.axis_index("s") * 2 + lax.axis_index("c")


def _sc_body(x_hbm, y_hbm, z_hbm, t0, t1, t2, t3, out, *scr):
    xv, yv, zv, hv, wv, st = scr[:6]
    per_level = []
    k = 6
    for _ in _LEVELS:
        per_level.append(scr[k:k + 12])
        k += 12
    sem = scr[k]

    wid = _worker_id()
    base = jnp.minimum(wid * _WPT, _N - _WPT)

    pltpu.sync_copy(x_hbm.at[pl.ds(base, _WPT)], xv)
    pltpu.sync_copy(y_hbm.at[pl.ds(base, _WPT)], yv)
    pltpu.sync_copy(z_hbm.at[pl.ds(base, _WPT)], zv)

    lane = lax.iota(jnp.int32, 16)
    is0 = lane == 0
    is1 = lane == 1

    def hw_body(c, carry):
        s = c * 16
        xx = xv[pl.ds(s, 16)]
        yy = yv[pl.ds(s, 16)]
        zz = zv[pl.ds(s, 16)]
        nz = -zz
        hh = 250.0 * (-yy) / nz + 112.0
        ww = 250.0 * xx / nz + 112.0
        hv[pl.ds(s, 16)] = jnp.minimum(jnp.maximum(hh, 0.0), 223.0)
        wv[pl.ds(s, 16)] = jnp.minimum(jnp.maximum(ww, 0.0), 223.0)
        return carry

    lax.fori_loop(0, _WPT // 16, hw_body, 0)

    tabs = (t0, t1, t2, t3)

    def blk_body(b, carry):
        pb = b * _BLK

        for g in range(_BLK // 16):
            xx = xv[pl.ds(pb + g * 16, 16)]
            yy = yv[pl.ds(pb + g * 16, 16)]
            zz = zv[pl.ds(pb + g * 16, 16)]
            for j in range(16):
                cvec = jnp.where(is0, xx[j], jnp.where(is1, yy[j], zz[j]))
                st[g * 16 + j, 7, pl.ds(64, 16)] = cvec

        for lvl, (H, dim, scale, col0, sub) in enumerate(_LEVELS):
            tab = tabs[lvl]
            q11, q21, q12, q22, i11, i21, i12, i22, w11, w21, w12, w22 = \
                per_level[lvl]
            for sb in range(_BLK // sub):
                ps = pb + sb * sub

                for c in range(sub // 16):
                    s = ps + c * 16
                    t = c * 16
                    hx = hv[pl.ds(s, 16)] * scale
                    wy = wv[pl.ds(s, 16)] * scale
                    x1i = hx.astype(jnp.int32)
                    x1f = x1i.astype(jnp.float32)
                    x2f = jnp.where(x1f == hx, x1f, x1f + 1.0)
                    x2i = jnp.minimum(x2f.astype(jnp.int32), H - 1)
                    y1i = wy.astype(jnp.int32)
                    y1f = y1i.astype(jnp.float32)
                    y2f = jnp.where(y1f == wy, y1f, y1f + 1.0)
                    y2i = jnp.minimum(y2f.astype(jnp.int32), H - 1)
                    dx2 = x2f - hx
                    dx1 = hx - x1f
                    dy2 = y2f - wy
                    dy1 = wy - y1f
                    i11[pl.ds(t, 16)] = x1i * H + y1i
                    i21[pl.ds(t, 16)] = x2i * H + y1i
                    i12[pl.ds(t, 16)] = x1i * H + y2i
                    i22[pl.ds(t, 16)] = x2i * H + y2i
                    w11[pl.ds(t, 16)] = dx2 * dy2
                    w21[pl.ds(t, 16)] = dx1 * dy2
                    w12[pl.ds(t, 16)] = dx2 * dy1
                    w22[pl.ds(t, 16)] = dx1 * dy1

                c1 = pltpu.async_copy(tab.at[i11], q11, sem)
                c2 = pltpu.async_copy(tab.at[i21], q21, sem)
                c3 = pltpu.async_copy(tab.at[i12], q12, sem)
                c4 = pltpu.async_copy(tab.at[i22], q22, sem)
                c1.wait()
                c2.wait()
                c3.wait()
                c4.wait()

                for g in range(sub // 16):
                    a16 = w11[pl.ds(g * 16, 16)]
                    b16 = w21[pl.ds(g * 16, 16)]
                    c16 = w12[pl.ds(g * 16, 16)]
                    d16 = w22[pl.ds(g * 16, 16)]
                    for j in range(16):
                        a, bw, cw, dw = a16[j], b16[j], c16[j], d16[j]
                        p = g * 16 + j
                        rp = sb * sub + g * 16 + j

                        def ch_body(kk, carry3, p=p, rp=rp, a=a, bw=bw,
                                    cw=cw, dw=dw, q11=q11, q21=q21,
                                    q12=q12, q22=q22, col0=col0):
                            d = pl.ds(kk * 16, 16)
                            v = (a * q11[p, d] + bw * q21[p, d]
                                 + cw * q12[p, d] + dw * q22[p, d])
                            cc = col0 + kk * 16
                            st[rp, lax.shift_right_logical(cc, 7),
                               pl.ds(lax.bitwise_and(cc, 127), 16)] = v
                            return carry3

                        lax.fori_loop(0, dim // 16, ch_body, 0)

        pltpu.sync_copy(st, out.at[pl.ds(base + pb, _BLK)])
        return carry

    lax.fori_loop(0, _WPT // _BLK, blk_body, 0)


@functools.cache
def _build_sc_kernel():
    mesh = plsc.VectorSubcoreMesh(
        core_axis_name="c", subcore_axis_name="s", num_cores=2, num_subcores=16
    )
    return functools.partial(
        pl.kernel,
        out_type=jax.ShapeDtypeStruct((_N, 8, 128), jnp.float32),
        mesh=mesh,
        scratch_types=_scratch_types(),
        compiler_params=pltpu.CompilerParams(use_tc_tiling_on_sc=False),
    )(_sc_body)


_FB = 1000


def _tc_finish_body(big_ref, out_ref):
    v = big_ref[...]
    out_ref[:, pl.ds(0, 3)] = v[:, 7, 64:67]
    for j in range(7):
        out_ref[:, pl.ds(3 + j * 128, 128)] = v[:, j, :]
    out_ref[:, pl.ds(3 + 896, 64)] = v[:, 7, :64]


@functools.cache
def _build_tc_finisher():
    return pl.pallas_call(
        _tc_finish_body,
        grid=(_N // _FB,),
        in_specs=[pl.BlockSpec((_FB, 8, 128), lambda i: (i, 0, 0))],
        out_specs=pl.BlockSpec((_FB, 963), lambda i: (i, 0)),
        out_shape=jax.ShapeDtypeStruct((_N, 963), jnp.float32),
    )


def kernel(inputs, img_feat0, img_feat1, img_feat2, img_feat3):
    x = inputs[:, 0]
    y = inputs[:, 1]
    z = inputs[:, 2]
    t0 = img_feat0.reshape(56 * 56, 64)
    t1 = img_feat1.reshape(28 * 28, 128)
    t2 = img_feat2.reshape(14 * 14, 256)
    t3 = img_feat3.reshape(7 * 7, 512)
    big = _build_sc_kernel()(x, y, z, t0, t1, t2, t3)
    return _build_tc_finisher()(big)
